# baseline (speedup 1.0000x reference)
import functools

import jax
import jax.numpy as jnp
from jax import lax
from jax.experimental import pallas as pl
from jax.experimental.pallas import tpu as pltpu
from jax.experimental.pallas import tpu_sc as plsc

HYP_DIM = 16
EMBED_DIM = 64
PAIR_DIM = 2 * EMBED_DIM
LANES = 16
NUM_CORES = 2
NUM_SUBCORES = 16
NUM_WORKERS = NUM_CORES * NUM_SUBCORES

MAX_NORM = 1.0 - 1e-5
MAX_NORM2 = MAX_NORM * MAX_NORM

CB = 16
NBUF = 2


@functools.lru_cache(maxsize=None)
def _make_kernel(bsz, feat):
  assert bsz % (NUM_WORKERS * CB) == 0
  bpw = bsz // NUM_WORKERS
  n_chunks = bpw // CB
  rows_per_chunk = CB * feat
  assert n_chunks % NBUF == 0

  mesh = plsc.VectorSubcoreMesh(
      core_axis_name="c", subcore_axis_name="s",
      num_cores=NUM_CORES, num_subcores=NUM_SUBCORES)

  @functools.partial(
      pl.kernel,
      out_type=jax.ShapeDtypeStruct((bsz, feat, EMBED_DIM), jnp.float32),
      mesh=mesh,
      scratch_types=[
          pltpu.VMEM((bpw, feat), jnp.int32),
          pltpu.VMEM((CB, feat, PAIR_DIM), jnp.float32),
          pltpu.VMEM((CB, feat, PAIR_DIM), jnp.float32),
          pltpu.VMEM((feat, CB), jnp.float32),
          pltpu.VMEM((feat, CB), jnp.float32),
          pltpu.SemaphoreType.DMA,
          pltpu.SemaphoreType.DMA,
          pltpu.SemaphoreType.DMA,
          pltpu.SemaphoreType.DMA,
          pltpu.SemaphoreType.DMA,
          pltpu.SemaphoreType.DMA,
      ],
      compiler_params=pltpu.CompilerParams(use_tc_tiling_on_sc=False),
  )
  def gather_project(pidx_hbm, par_hbm, table_hbm, out_hbm,
                     idx_v, r0, r1, p0, p1, g0, g1, s0, s1, q0, q1):
    bufs = (r0, r1)
    pars = (p0, p1)
    gsems = (g0, g1)
    ssems = (s0, s1)
    psems = (q0, q1)

    wid = lax.axis_index("s") * NUM_CORES + lax.axis_index("c")
    b0 = wid * bpw
    pltpu.sync_copy(pidx_hbm.at[pl.ds(b0, bpw)], idx_v)

    iota = lax.iota(jnp.int32, LANES)
    lane_lo = iota < 8
    zeros_i = lax.broadcast(jnp.int32(0), (LANES,))
    eights_i = lax.broadcast(jnp.int32(8), (LANES,))
    ones_f = lax.broadcast(jnp.float32(1.0), (LANES,))
    cap_f = lax.broadcast(jnp.float32(1e12), (LANES,))
    _dnums = lax.GatherDimensionNumbers(
        offset_dims=(), collapsed_slice_dims=(0,), start_index_map=(0,))

    def permute(x, idx):
      return lax.gather(x, idx[:, None], _dnums, slice_sizes=(1,),
                        mode=lax.GatherScatterMode.PROMISE_IN_BOUNDS)

    def issue_gather(k, b):
      pltpu.async_copy(
          par_hbm.at[:, pl.ds(b0 + k * CB, CB)], pars[b], psems[b])
      for j in range(CB):
        pltpu.async_copy(
            table_hbm.at[idx_v.at[k * CB + j]],
            bufs[b].at[j],
            gsems[b])

    def wait_gather(b):
      pltpu.make_async_copy(
          par_hbm.at[:, pl.ds(0, CB)], pars[b], psems[b]).wait()
      for j in range(CB):
        pltpu.make_async_copy(
            table_hbm.at[pl.ds(0, feat)], bufs[b].at[j], gsems[b]).wait()

    def issue_store(k, b):
      pltpu.async_copy(
          bufs[b].at[:, :, pl.ds(0, EMBED_DIM)],
          out_hbm.at[pl.ds(b0 + k * CB, CB)], ssems[b])

    def wait_store(b):
      pltpu.make_async_copy(
          bufs[b].at[:, :, pl.ds(0, EMBED_DIM)],
          out_hbm.at[pl.ds(0, CB)], ssems[b]).wait()

    def compute(b):
      rows_v = bufs[b]
      par_v = pars[b]

      @plsc.parallel_loop(0, rows_per_chunk, unroll=4)
      def _row(q):
        i = q & (CB - 1)
        f = lax.shift_right_logical(q, 4)
        p = permute(par_v[f], lax.broadcast(i, (LANES,)))
        hl = rows_v[i, f, pl.ds(0, 16)]
        a0 = rows_v[i, f, pl.ds(16, 16)]
        a1 = rows_v[i, f, pl.ds(32, 16)]
        a2 = rows_v[i, f, pl.ds(48, 16)]
        hh = rows_v[i, f, pl.ds(64, 16)]
        b1 = rows_v[i, f, pl.ds(80, 16)]
        b2 = rows_v[i, f, pl.ds(96, 16)]
        b3 = rows_v[i, f, pl.ds(112, 16)]
        h = hl + p * (hh - hl)
        t0 = a0 + p * (b1 - a0)
        t1 = a1 + p * (b2 - a1)
        t2 = a2 + p * (b3 - a2)
        u = h * h
        v = t0 * t0 + t1 * t1 + t2 * t2
        a = u + permute(u, iota ^ 8)
        c = v + permute(v, iota ^ 8)
        z = jnp.where(lane_lo, a, c)
        for d in (4, 2, 1):
          z = z + permute(z, iota ^ d)
        w = lax.bitcast_convert_type(z, jnp.int32)
        w = jnp.int32(0x5F3759DF) - lax.shift_right_logical(w, 1)
        y = lax.bitcast_convert_type(w, jnp.float32)
        zh = z * jnp.float32(0.5)
        y = y * (jnp.float32(1.5) - zh * y * y)
        y = y * (jnp.float32(1.5) - zh * y * y)
        hsel = jnp.where(z > jnp.float32(MAX_NORM2),
                         y * jnp.float32(MAX_NORM), ones_f)
        hscale = permute(hsel, zeros_i)
        sinv = permute(jnp.minimum(y, cap_f), eights_i)
        rows_v[i, f, pl.ds(0, 16)] = h * hscale
        rows_v[i, f, pl.ds(16, 16)] = t0 * sinv
        rows_v[i, f, pl.ds(32, 16)] = t1 * sinv
        rows_v[i, f, pl.ds(48, 16)] = t2 * sinv

    issue_gather(0, 0)

    @pl.loop(0, n_chunks, step=NBUF)
    def _pipe(cbase):
      for d in range(NBUF):
        k = cbase + d
        b = d % NBUF
        o = (d + 1) % NBUF
        wait_gather(b)

        @pl.when(k > 0)
        def _():
          wait_store(o)

        @pl.when(k + 1 < n_chunks)
        def _():
          issue_gather(k + 1, o)
        compute(b)
        issue_store(k, b)

    wait_store((n_chunks - 1) % NBUF)

  return gather_project


@jax.jit
def kernel(indices, table):
  bsz, feat = indices.shape
  idx32 = indices.astype(jnp.int32)
  pidx = idx32 >> 1
  parity_t = (idx32 & 1).astype(jnp.float32).T
  table2 = table.reshape(table.shape[0] // 2, PAIR_DIM)
  return _make_kernel(bsz, feat)(pidx, parity_t, table2)


# speedup vs baseline: 1.1024x; 1.1024x over previous
import functools

import jax
import jax.numpy as jnp
from jax import lax
from jax.experimental import pallas as pl
from jax.experimental.pallas import tpu as pltpu
from jax.experimental.pallas import tpu_sc as plsc

HYP_DIM = 16
EMBED_DIM = 64
LANES = 16
NUM_CORES = 2
NUM_SUBCORES = 16
NUM_WORKERS = NUM_CORES * NUM_SUBCORES

MAX_NORM = 1.0 - 1e-5
MAX_NORM2 = MAX_NORM * MAX_NORM

CB = 16
NBUF = 3


@functools.lru_cache(maxsize=None)
def _make_kernel(bsz, feat):
  assert bsz % (NUM_WORKERS * CB) == 0
  bpw = bsz // NUM_WORKERS
  n_chunks = bpw // CB
  rows_per_chunk = CB * feat
  assert n_chunks % NBUF == 2

  mesh = plsc.VectorSubcoreMesh(
      core_axis_name="c", subcore_axis_name="s",
      num_cores=NUM_CORES, num_subcores=NUM_SUBCORES)

  @functools.partial(
      pl.kernel,
      out_type=jax.ShapeDtypeStruct((bsz, feat, EMBED_DIM), jnp.float32),
      mesh=mesh,
      scratch_types=[
          pltpu.VMEM((bpw, feat), jnp.int32),
          pltpu.VMEM((CB, feat, EMBED_DIM), jnp.float32),
          pltpu.VMEM((CB, feat, EMBED_DIM), jnp.float32),
          pltpu.VMEM((CB, feat, EMBED_DIM), jnp.float32),
          pltpu.SemaphoreType.DMA,
          pltpu.SemaphoreType.DMA,
          pltpu.SemaphoreType.DMA,
          pltpu.SemaphoreType.DMA,
          pltpu.SemaphoreType.DMA,
          pltpu.SemaphoreType.DMA,
      ],
      compiler_params=pltpu.CompilerParams(use_tc_tiling_on_sc=False),
  )
  def gather_project(idx_hbm, table_hbm, out_hbm,
                     idx_v, r0, r1, r2, g0, g1, g2, s0, s1, s2):
    bufs = (r0, r1, r2)
    gsems = (g0, g1, g2)
    ssems = (s0, s1, s2)

    wid = lax.axis_index("s") * NUM_CORES + lax.axis_index("c")
    b0 = wid * bpw
    pltpu.sync_copy(idx_hbm.at[pl.ds(b0, bpw)], idx_v)

    iota = lax.iota(jnp.int32, LANES)
    lane_lo = iota < 8
    zeros_i = lax.broadcast(jnp.int32(0), (LANES,))
    eights_i = lax.broadcast(jnp.int32(8), (LANES,))
    ones_f = lax.broadcast(jnp.float32(1.0), (LANES,))
    cap_f = lax.broadcast(jnp.float32(1e12), (LANES,))
    _dnums = lax.GatherDimensionNumbers(
        offset_dims=(), collapsed_slice_dims=(0,), start_index_map=(0,))

    def permute(x, idx):
      return lax.gather(x, idx[:, None], _dnums, slice_sizes=(1,),
                        mode=lax.GatherScatterMode.PROMISE_IN_BOUNDS)

    def issue_gather(k, b):
      for j in range(CB):
        pltpu.async_copy(
            table_hbm.at[idx_v.at[k * CB + j]],
            bufs[b].at[j],
            gsems[b])

    def wait_gather(b):
      for j in range(CB):
        pltpu.make_async_copy(
            table_hbm.at[pl.ds(0, feat)], bufs[b].at[j], gsems[b]).wait()

    def issue_store(k, b):
      pltpu.async_copy(
          bufs[b], out_hbm.at[pl.ds(b0 + k * CB, CB)], ssems[b])

    def wait_store(b):
      pltpu.make_async_copy(
          bufs[b], out_hbm.at[pl.ds(0, CB)], ssems[b]).wait()

    def compute(b):
      rows_v = bufs[b]

      @plsc.parallel_loop(0, rows_per_chunk, unroll=8)
      def _row(q):
        i = q & (CB - 1)
        f = lax.shift_right_logical(q, 4)
        h = rows_v[i, f, pl.ds(0, 16)]
        t0 = rows_v[i, f, pl.ds(16, 16)]
        t1 = rows_v[i, f, pl.ds(32, 16)]
        t2 = rows_v[i, f, pl.ds(48, 16)]
        u = h * h
        v = t0 * t0 + t1 * t1 + t2 * t2
        a = u + permute(u, iota ^ 8)
        c = v + permute(v, iota ^ 8)
        z = jnp.where(lane_lo, a, c)
        for d in (4, 2, 1):
          z = z + permute(z, iota ^ d)
        w = lax.bitcast_convert_type(z, jnp.int32)
        w = jnp.int32(0x5F3759DF) - lax.shift_right_logical(w, 1)
        y = lax.bitcast_convert_type(w, jnp.float32)
        zh = z * jnp.float32(0.5)
        y = y * (jnp.float32(1.5) - zh * y * y)
        y = y * (jnp.float32(1.5) - zh * y * y)
        hsel = jnp.where(z > jnp.float32(MAX_NORM2),
                         y * jnp.float32(MAX_NORM), ones_f)
        hscale = permute(hsel, zeros_i)
        sinv = permute(jnp.minimum(y, cap_f), eights_i)
        rows_v[i, f, pl.ds(0, 16)] = h * hscale
        rows_v[i, f, pl.ds(16, 16)] = t0 * sinv
        rows_v[i, f, pl.ds(32, 16)] = t1 * sinv
        rows_v[i, f, pl.ds(48, 16)] = t2 * sinv

    issue_gather(0, 0)
    issue_gather(1, 1)
    wait_gather(0)
    compute(0)
    issue_gather(2, 2)
    issue_store(0, 0)

    @pl.loop(1, n_chunks - 3, step=NBUF)
    def _pipe(cbase):
      for d in range(NBUF):
        k = cbase + d
        b = (1 + d) % NBUF
        b2 = (b + 2) % NBUF
        wait_gather(b)
        compute(b)
        wait_store(b2)
        if d < NBUF - 1:
          issue_gather(k + 2, b2)
        else:
          @pl.when(cbase + NBUF + 1 < n_chunks)
          def _():
            issue_gather(k + 2, b2)
        issue_store(k, b)

    kl = n_chunks - 1
    bl = kl % NBUF
    b2l = (bl + 2) % NBUF
    wait_gather(bl)
    compute(bl)
    wait_store(b2l)
    issue_store(kl, bl)
    wait_store(bl)

  return gather_project


@jax.jit
def kernel(indices, table):
  bsz, feat = indices.shape
  return _make_kernel(bsz, feat)(indices.astype(jnp.int32), table)
